# baseline (speedup 1.0000x reference)
import functools

import jax
import jax.numpy as jnp
from jax import lax
from jax.experimental import pallas as pl
from jax.experimental.pallas import tpu as pltpu
from jax.experimental.pallas import tpu_sc as plsc

_N = 50000
_H = 64
_HW = 128
_NE = 200000
_NK = 8
_VOFF = _NK * _N
_HALF = 25000
_HPAIR = _HALF // 2
_ACCR = 12544
_TRASH = 12500
_CH = 64
_NCHUNK = 25024
_NPAD = _NCHUNK * _CH
_PT = _NCHUNK // 16
_ZR = 784

_NPN = 57344
_EPL = _NPN // 128 // 32
_EPT = _NPN * 10 // 128 // 32

_LAYER_TIMESTEPS = (3, 1, 3, 1)
_RESIDUALS = {1: (0,), 3: (0, 1)}

@functools.cache
def _sc_mesh():
    return plsc.VectorSubcoreMesh(
        core_axis_name="c", subcore_axis_name="s",
        num_cores=2, num_subcores=16)


def _msgs_body(y_hbm, g_hbm, s_hbm, z_hbm, out_hbm,
               gidx0, sidx0, rows0, gidx1, sidx1, rows1, acc, sem0, sem1):
    c = lax.axis_index("c")
    s = lax.axis_index("s")

    zb = s * _ZR
    pltpu.sync_copy(z_hbm, acc.at[pl.ds(zb, _ZR)])
    plsc.subcore_barrier()

    def _fire(i, gidx, sidx, rows, sem):
        gch = (c * _NCHUNK + s * _PT + i) * _CH
        pltpu.sync_copy(g_hbm.at[pl.ds(gch, _CH)], gidx)
        pltpu.sync_copy(s_hbm.at[pl.ds(gch, _CH)], sidx)
        pltpu.async_copy(y_hbm.at[gidx], rows, sem)

    def _drain(i, gidx, sidx, rows, sem):
        pltpu.make_async_copy(y_hbm.at[gidx], rows, sem).wait()
        pltpu.sync_copy(rows, acc.at[sidx], add=True)

    _fire(0, gidx0, sidx0, rows0, sem0)

    def _pair(k, carry):
        i0 = k * 2
        _fire(i0 + 1, gidx1, sidx1, rows1, sem1)
        _drain(i0, gidx0, sidx0, rows0, sem0)
        _fire(i0 + 2, gidx0, sidx0, rows0, sem0)
        _drain(i0 + 1, gidx1, sidx1, rows1, sem1)
        return carry
    lax.fori_loop(0, _PT // 2 - 1, _pair, 0)
    _fire(_PT - 1, gidx1, sidx1, rows1, sem1)
    _drain(_PT - 2, gidx0, sidx0, rows0, sem0)
    _drain(_PT - 1, gidx1, sidx1, rows1, sem1)

    plsc.subcore_barrier()

    ob = s * _ZR
    pltpu.sync_copy(acc.at[pl.ds(ob, _ZR)],
                    out_hbm.at[pl.ds(c * _ACCR + ob, _ZR)])


@functools.cache
def _msgs_kernel():
    return functools.partial(
        pl.kernel,
        out_type=jax.ShapeDtypeStruct((2 * _ACCR, _HW), jnp.float32),
        mesh=_sc_mesh(),
        scratch_types=[
            pltpu.VMEM((_CH,), jnp.int32),
            pltpu.VMEM((_CH,), jnp.int32),
            pltpu.VMEM((_CH, _HW), jnp.float32),
            pltpu.VMEM((_CH,), jnp.int32),
            pltpu.VMEM((_CH,), jnp.int32),
            pltpu.VMEM((_CH, _HW), jnp.float32),
            pltpu.VMEM_SHARED((_ACCR, _HW), jnp.float32),
            pltpu.SemaphoreType.DMA,
            pltpu.SemaphoreType.DMA,
        ],
    )(_msgs_body)


def _embed_body(lab_hbm, typ_hbm, tok_hbm, tidx_hbm, x1_hbm, x2_hbm,
                idx0, buf0, idx1, buf1, sem0, sem1):
    c = lax.axis_index("c")
    s = lax.axis_index("s")
    w = c * 16 + s

    def _stream(idx_hbm_, src, out, per):
        base = w * per

        def _fire(i, idxv, buf, sem):
            pltpu.sync_copy(idx_hbm_.at[pl.ds((base + i) * 128, 128)], idxv)
            pltpu.async_copy(src.at[idxv], buf, sem)

        def _drain(i, idxv, buf, sem):
            pltpu.make_async_copy(src.at[idxv], buf, sem).wait()
            pltpu.sync_copy(buf, out.at[pl.ds((base + i) * 128, 128)])

        _fire(0, idx0, buf0, sem0)

        def _pair(k, carry):
            i0 = k * 2
            _fire(i0 + 1, idx1, buf1, sem1)
            _drain(i0, idx0, buf0, sem0)
            _fire(i0 + 2, idx0, buf0, sem0)
            _drain(i0 + 1, idx1, buf1, sem1)
            return carry
        lax.fori_loop(0, per // 2 - 1, _pair, 0)
        _fire(per - 1, idx1, buf1, sem1)
        _drain(per - 2, idx0, buf0, sem0)
        _drain(per - 1, idx1, buf1, sem1)

    _stream(tok_hbm, lab_hbm, x1_hbm, _EPL)
    _stream(tidx_hbm, typ_hbm, x2_hbm, _EPT)


@functools.cache
def _embed_kernel():
    return functools.partial(
        pl.kernel,
        out_type=(jax.ShapeDtypeStruct((_NPN, 128), jnp.float32),
                  jax.ShapeDtypeStruct((10 * _NPN, 128), jnp.float32)),
        mesh=_sc_mesh(),
        scratch_types=[
            pltpu.VMEM((128,), jnp.int32),
            pltpu.VMEM((128, 128), jnp.float32),
            pltpu.VMEM((128,), jnp.int32),
            pltpu.VMEM((128, 128), jnp.float32),
            pltpu.SemaphoreType.DMA,
            pltpu.SemaphoreType.DMA,
        ],
    )(_embed_body)


_BR = 1000
_GRID = _N // _BR


def _row_spec():
    return pl.BlockSpec((_BR, _H), lambda i: (i, 0))


def _y_spec():
    return pl.BlockSpec((2, _NK, _BR, _HW), lambda i: (0, 0, i, 0))


def _full(shape):
    return pl.BlockSpec(shape, lambda i: tuple(0 for _ in shape))


def _edge_transforms(x, w_ref, y_ref):
    z = jnp.zeros((x.shape[0], _H), jnp.float32)
    for k in range(_NK):
        d = jnp.dot(x, w_ref[k], preferred_element_type=jnp.float32)
        y_ref[0, k] = jnp.concatenate([d, z], axis=1)
        y_ref[1, k] = jnp.concatenate([z, d], axis=1)


def _init_body(x1_ref, t_ref, m_ref, w1_ref, w2_ref, w_ref, h_ref, y_ref):
    m = m_ref[...]
    acc = t_ref[0][:, :32] * m[:, 0:1]
    for j in range(1, 10):
        acc = acc + t_ref[j][:, :32] * m[:, j:j + 1]
    x2 = acc / jnp.maximum(jnp.sum(m, axis=1, keepdims=True), 1e-6)
    h = (jnp.dot(x1_ref[:, :32], w1_ref[...],
                 preferred_element_type=jnp.float32)
         + jnp.dot(x2, w2_ref[...], preferred_element_type=jnp.float32))
    h_ref[...] = h
    _edge_transforms(h, w_ref, y_ref)


_init_call = pl.pallas_call(
    _init_body,
    grid=(_GRID,),
    in_specs=[pl.BlockSpec((_BR, 128), lambda i: (i, 0)),
              pl.BlockSpec((10, _BR, 128), lambda i: (0, i, 0)),
              pl.BlockSpec((_BR, 10), lambda i: (i, 0)),
              _full((32, _H)), _full((32, _H)), _full((_NK, _H, _H))],
    out_specs=[_row_spec(), _y_spec()],
    out_shape=[jax.ShapeDtypeStruct((_N, _H), jnp.float32),
               jax.ShapeDtypeStruct((2, _NK, _N, _HW), jnp.float32)],
)


def _yonly_body(cur_ref, w_ref, y_ref):
    _edge_transforms(cur_ref[...], w_ref, y_ref)


_yonly_call = pl.pallas_call(
    _yonly_body,
    grid=(_GRID,),
    in_specs=[_row_spec(), _full((_NK, _H, _H))],
    out_specs=_y_spec(),
    out_shape=jax.ShapeDtypeStruct((2, _NK, _N, _HW), jnp.float32),
)


def _res2_body(a_ref, b_ref, w_ref, c_ref, y_ref):
    x = a_ref[...] + b_ref[...]
    c_ref[...] = x
    _edge_transforms(x, w_ref, y_ref)


_res2_call = pl.pallas_call(
    _res2_body,
    grid=(_GRID,),
    in_specs=[_row_spec(), _row_spec(), _full((_NK, _H, _H))],
    out_specs=[_row_spec(), _y_spec()],
    out_shape=[jax.ShapeDtypeStruct((_N, _H), jnp.float32),
               jax.ShapeDtypeStruct((2, _NK, _N, _HW), jnp.float32)],
)


def _res3_body(a_ref, b_ref, d_ref, w_ref, c_ref, y_ref):
    x = a_ref[...] + b_ref[...] + d_ref[...]
    c_ref[...] = x
    _edge_transforms(x, w_ref, y_ref)


_res3_call = pl.pallas_call(
    _res3_body,
    grid=(_GRID,),
    in_specs=[_row_spec(), _row_spec(), _row_spec(), _full((_NK, _H, _H))],
    out_specs=[_row_spec(), _y_spec()],
    out_shape=[jax.ShapeDtypeStruct((_N, _H), jnp.float32),
               jax.ShapeDtypeStruct((2, _NK, _N, _HW), jnp.float32)],
)


def _gru_body(m_ref, h_ref, wm_ref, uh_ref, b_ref, o_ref):
    m = m_ref[...]
    h = h_ref[...]
    gm = jnp.dot(m, wm_ref[...], preferred_element_type=jnp.float32)
    gh = jnp.dot(h, uh_ref[...], preferred_element_type=jnp.float32)
    b = b_ref[...]
    z = jax.nn.sigmoid(gm[:, :_H] + gh[:, :_H] + b[:, :_H])
    r = jax.nn.sigmoid(gm[:, _H:2 * _H] + gh[:, _H:2 * _H] + b[:, _H:2 * _H])
    ht = jnp.tanh(gm[:, 2 * _H:] + r * gh[:, 2 * _H:] + b[:, 2 * _H:])
    o_ref[...] = (1.0 - z) * h + z * ht


_gru_call = pl.pallas_call(
    _gru_body,
    grid=(_GRID,),
    in_specs=[_row_spec(), _row_spec(),
              _full((_H, 3 * _H)), _full((_H, 3 * _H)), _full((1, 3 * _H))],
    out_specs=_row_spec(),
    out_shape=jax.ShapeDtypeStruct((_N, _H), jnp.float32),
)


def _build_streams(adjs):
    gs, ss = [], []
    for t in range(4):
        src = adjs[t][:, 0]
        dst = adjs[t][:, 1]
        gs += [src, dst]
        ss += [dst, src]
    g = jnp.stack(gs)
    s = jnp.stack(ss)
    gf = (jnp.arange(_NK, dtype=jnp.int32)[:, None] * _N + g).reshape(-1)
    sf = s.reshape(-1)
    padlen = _NPAD - _NK * _NE
    gf = jnp.concatenate([gf, jnp.zeros((padlen,), jnp.int32)])
    sf = jnp.concatenate([sf, jnp.full((padlen,), _N, jnp.int32)])
    own0 = sf < _HALF
    own1 = (sf >= _HALF) & (sf < _N)
    l1 = sf - _HALF
    g0 = jnp.where(own0, (sf & 1) * _VOFF + gf, 0)
    g1 = jnp.where(own1, (l1 & 1) * _VOFF + gf, 0)
    s0 = jnp.where(own0, sf >> 1, _TRASH)
    s1 = jnp.where(own1, l1 >> 1, _TRASH)
    return (jnp.concatenate([g0, g1]), jnp.concatenate([s0, s1]))


def kernel(cg_node_label_token_ids, cg_node_type_ids, cg_node_type_ids_mask,
           adj_e0, adj_e1, adj_e2, adj_e3,
           label_embeddings, type_embeddings, W_init,
           edge_W, gru_Wm, gru_Uh, gru_b):
    adjs = [a.astype(jnp.int32) for a in (adj_e0, adj_e1, adj_e2, adj_e3)]
    g_hbm, s_hbm = _build_streams(adjs)

    tok = cg_node_label_token_ids.astype(jnp.int32)
    tids = cg_node_type_ids.astype(jnp.int32)
    mask = cg_node_type_ids_mask
    pad_n = _NPN - _N
    tok_p = jnp.concatenate([tok, jnp.zeros((pad_n,), jnp.int32)])
    tidT = jnp.pad(tids, ((0, pad_n), (0, 0))).T.reshape(-1)

    lab_p = jnp.pad(label_embeddings, ((0, 0), (0, 96)))
    typ_p = jnp.pad(type_embeddings, ((0, 0), (0, 96)))
    x1, x2raw = _embed_kernel()(lab_p, typ_p, tok_p, tidT)
    traw = x2raw.reshape(10, _NPN, 128)
    zrows = jnp.zeros((_ZR, _HW), jnp.float32)

    w1 = W_init[:32]
    w2 = W_init[32:]
    gru_b2 = gru_b.reshape(len(_LAYER_TIMESTEPS), 1, 3 * _H)

    cur, y = _init_call(x1, traw, mask, w1, w2, edge_W[0])
    states = [cur]

    for l, n_steps in enumerate(_LAYER_TIMESTEPS):
        if l > 0:
            res = _RESIDUALS.get(l, ())
            if len(res) == 2:
                cur, y = _res3_call(states[-1], states[res[0]],
                                    states[res[1]], edge_W[l])
            elif len(res) == 1:
                cur, y = _res2_call(states[-1], states[res[0]], edge_W[l])
            else:
                cur = states[-1]
                y = _yonly_call(cur, edge_W[l])
        for step in range(n_steps):
            mraw = _msgs_kernel()(y.reshape(2 * _NK * _N, _HW), g_hbm,
                                  s_hbm, zrows)
            msgs = (mraw.reshape(2, _ACCR, 2, _H)[:, :_HPAIR]
                    .reshape(_N, _H))
            cur = _gru_call(msgs, cur, gru_Wm[l], gru_Uh[l], gru_b2[l])
            if step + 1 < n_steps:
                y = _yonly_call(cur, edge_W[l])
        states.append(cur)

    return states[-1]
